# baseline (speedup 1.0000x reference)
import jax
import jax.numpy as jnp
from jax.experimental import pallas as pl
from jax.experimental.pallas import tpu as pltpu

_IN = 1024
_OUT = 32768
_K = 256
_B = 8
_BLOCK_N = 2048
_NBLK = _OUT // _BLOCK_N


def _float_key(t):
    i = jax.lax.bitcast_convert_type(t, jnp.int32)
    return jnp.where(i >= 0, i, i ^ jnp.int32(0x7FFFFFFF))


def _count_ge_tree(key, cand):
    m = (key >= cand).astype(jnp.int32)
    n = m.shape[1]
    parts = [jnp.sum(m[:, i * (n // 8):(i + 1) * (n // 8)], axis=1,
                     keepdims=True) for i in range(8)]
    return ((parts[0] + parts[1]) + (parts[2] + parts[3])) + \
           ((parts[4] + parts[5]) + (parts[6] + parts[7]))


def _wta_kernel(x_ref, w_ref, b_ref, out_ref, t_ref, th_ref, mb_ref):
    step = pl.program_id(0)
    t_blk = jax.lax.dot_general(
        x_ref[...], w_ref[...],
        (((1,), (1,)), ((), ())),
        preferred_element_type=jnp.float32,
    ) + b_ref[...]
    t_ref[:, pl.ds(step * _BLOCK_N, _BLOCK_N)] = t_blk

    @pl.when(step == _NBLK - 1)
    def _():
        t = t_ref[...]
        key = _float_key(t)

        msb = jnp.int32(-2147483648)
        sub = key[:, :512]
        est_u = jnp.zeros((_B, 1), jnp.int32)
        for bit in range(31, 13, -1):
            bitval = (1 << bit) if bit < 31 else -(1 << 31)
            cand_u = est_u | jnp.int32(bitval)
            cnt = jnp.sum((sub >= (cand_u ^ msb)).astype(jnp.int32),
                          axis=1, keepdims=True)
            est_u = jnp.where(cnt >= 4, cand_u, est_u)
        base = ((est_u ^ msb) & jnp.int32(-(1 << 14))) - jnp.int32(1 << 16)
        c_lo = _count_ge_tree(key, base)
        c_hi = _count_ge_tree(key, base + jnp.int32(1 << 17))
        good = jnp.all((c_lo >= _K) & (c_hi < _K))

        @pl.when(good)
        def _refine():
            prefix = base
            for bit in range(16, -1, -1):
                cand = prefix + jnp.int32(1 << bit)
                cnt = _count_ge_tree(key, cand)
                prefix = jnp.where(cnt >= _K, cand, prefix)
            th_ref[...] = jnp.broadcast_to(prefix, (_B, 128))

        @pl.when(jnp.logical_not(good))
        def _full():
            prefix_u = jnp.zeros((_B, 1), jnp.int32)
            for bit in range(31, -1, -1):
                bitval = (1 << bit) if bit < 31 else -(1 << 31)
                cand_u = prefix_u | jnp.int32(bitval)
                cnt = _count_ge_tree(key, cand_u ^ msb)
                prefix_u = jnp.where(cnt >= _K, cand_u, prefix_u)
            th_ref[...] = jnp.broadcast_to(prefix_u ^ msb, (_B, 128))

        thresh = th_ref[:, :1]

        gt = key > thresh
        eq = key == thresh
        n_ge = jnp.sum((gt | eq).astype(jnp.int32), axis=1, keepdims=True)

        col = jax.lax.broadcasted_iota(jnp.int32, (_B, _OUT), 1)
        mb_ref[...] = jnp.full((_B, 128), _OUT, jnp.int32)

        @pl.when(jnp.any(n_ge > _K))
        def _tie():
            r = _K - (n_ge - jnp.sum(eq.astype(jnp.int32), axis=1,
                                     keepdims=True))
            mpref = jnp.zeros((_B, 1), jnp.int32)
            for bit in range(15, -1, -1):
                cand = mpref | jnp.int32(1 << bit)
                cntc = jnp.sum((eq & (col < cand)).astype(jnp.int32),
                               axis=1, keepdims=True)
                mpref = jnp.where(cntc <= r, cand, mpref)
            mb_ref[...] = jnp.broadcast_to(mpref, (_B, 128))

        accept = gt | (eq & (col < mb_ref[:, :1]))
        neg = jnp.float32(-jnp.inf)
        pooled = jnp.max(jnp.where(accept, t, neg), axis=0, keepdims=True)
        out_ref[...] = jnp.where(pooled == neg, jnp.float32(0.0), pooled)


def kernel(inputs, W, b):
    out = pl.pallas_call(
        _wta_kernel,
        grid=(_NBLK,),
        in_specs=[
            pl.BlockSpec((_B, _IN), lambda i: (0, 0)),
            pl.BlockSpec((_BLOCK_N, _IN), lambda i: (i, 0)),
            pl.BlockSpec((1, _BLOCK_N), lambda i: (0, i)),
        ],
        out_specs=pl.BlockSpec((1, _OUT), lambda i: (0, 0)),
        out_shape=jax.ShapeDtypeStruct((1, _OUT), jnp.float32),
        scratch_shapes=[pltpu.VMEM((_B, _OUT), jnp.float32),
                        pltpu.VMEM((_B, 128), jnp.int32),
                        pltpu.VMEM((_B, 128), jnp.int32)],
    )(inputs, W, b.reshape(1, _OUT))
    return out.reshape(_OUT)


# speedup vs baseline: 1.0403x; 1.0403x over previous
import jax
import jax.numpy as jnp
from jax.experimental import pallas as pl
from jax.experimental.pallas import tpu as pltpu

_IN = 1024
_OUT = 32768
_K = 256
_B = 8
_BLOCK_N = 2048
_NBLK = _OUT // _BLOCK_N


def _float_key(t):
    i = jax.lax.bitcast_convert_type(t, jnp.int32)
    return jnp.where(i >= 0, i, i ^ jnp.int32(0x7FFFFFFF))


def _key_float(k):
    i = jnp.where(k >= 0, k, k ^ jnp.int32(0x7FFFFFFF))
    return jax.lax.bitcast_convert_type(i, jnp.float32)


def _count_ge_tree(arr, cand):
    m = (arr >= cand).astype(jnp.int32)
    n = m.shape[1]
    p = [jnp.sum(m[:, i * (n // 16):(i + 1) * (n // 16)], axis=1,
                 keepdims=True) for i in range(16)]
    q = [(p[2 * i] + p[2 * i + 1]) for i in range(8)]
    return ((q[0] + q[1]) + (q[2] + q[3])) + ((q[4] + q[5]) + (q[6] + q[7]))


def _wta_kernel(x_ref, w_ref, b_ref, out_ref, t_ref, th_ref):
    step = pl.program_id(0)
    t_blk = jax.lax.dot_general(
        x_ref[...], w_ref[...],
        (((1,), (1,)), ((), ())),
        preferred_element_type=jnp.float32,
    ) + b_ref[...]
    t_ref[:, pl.ds(step * _BLOCK_N, _BLOCK_N)] = t_blk

    @pl.when(step == _NBLK - 1)
    def _():
        t = t_ref[...]
        msb = jnp.int32(-2147483648)

        prefix_u = jnp.zeros((_B, 1), jnp.int32)
        for bit in range(31, -1, -1):
            bitval = (1 << bit) if bit < 31 else -(1 << 31)
            cand_u = prefix_u | jnp.int32(bitval)
            cnt = _count_ge_tree(t, _key_float(cand_u ^ msb))
            prefix_u = jnp.where(cnt >= _K, cand_u, prefix_u)
        th_ref[...] = jnp.broadcast_to(prefix_u ^ msb, (_B, 128))

        tf = _key_float(th_ref[:, :1])
        ge = t >= tf
        n_ge = _count_ge_tree(t, tf)
        col = jax.lax.broadcasted_iota(jnp.int32, (_B, _OUT), 1)
        neg = jnp.float32(-jnp.inf)

        pooled = jnp.max(jnp.where(ge, t, neg), axis=0, keepdims=True)
        out_ref[...] = jnp.where(pooled == neg, jnp.float32(0.0), pooled)

        @pl.when(jnp.any(n_ge > _K))
        def _tie():
            gt = t > tf
            eq = ge & jnp.logical_not(gt)
            r = _K - (n_ge - jnp.sum(eq.astype(jnp.int32), axis=1,
                                     keepdims=True))
            mpref = jnp.zeros((_B, 1), jnp.int32)
            for bit in range(15, -1, -1):
                cand = mpref | jnp.int32(1 << bit)
                cntc = jnp.sum((eq & (col < cand)).astype(jnp.int32),
                               axis=1, keepdims=True)
                mpref = jnp.where(cntc <= r, cand, mpref)
            accept = gt | (eq & (col < mpref))
            pooled2 = jnp.max(jnp.where(accept, t, neg), axis=0,
                              keepdims=True)
            out_ref[...] = jnp.where(pooled2 == neg, jnp.float32(0.0),
                                     pooled2)


def kernel(inputs, W, b):
    out = pl.pallas_call(
        _wta_kernel,
        grid=(_NBLK,),
        in_specs=[
            pl.BlockSpec((_B, _IN), lambda i: (0, 0)),
            pl.BlockSpec((_BLOCK_N, _IN), lambda i: (i, 0)),
            pl.BlockSpec((1, _BLOCK_N), lambda i: (0, i)),
        ],
        out_specs=pl.BlockSpec((1, _OUT), lambda i: (0, 0)),
        out_shape=jax.ShapeDtypeStruct((1, _OUT), jnp.float32),
        scratch_shapes=[
            pltpu.VMEM((_B, _OUT), jnp.float32),
            pltpu.VMEM((_B, 128), jnp.int32),
        ],
    )(inputs, W, b.reshape(1, _OUT))
    return out.reshape(_OUT)


# speedup vs baseline: 1.0469x; 1.0064x over previous
import jax
import jax.numpy as jnp
from jax.experimental import pallas as pl
from jax.experimental.pallas import tpu as pltpu

_IN = 1024
_OUT = 32768
_K = 256
_B = 8
_BLOCK_N = 2048
_NBLK = _OUT // _BLOCK_N


def _float_key(t):
    i = jax.lax.bitcast_convert_type(t, jnp.int32)
    return jnp.where(i >= 0, i, i ^ jnp.int32(0x7FFFFFFF))


def _key_float(k):
    i = jnp.where(k >= 0, k, k ^ jnp.int32(0x7FFFFFFF))
    return jax.lax.bitcast_convert_type(i, jnp.float32)


def _count_ge_tree(arr, cand):
    m = (arr >= cand).astype(jnp.int32)
    n = m.shape[1]
    p = [jnp.sum(m[:, i * (n // 8):(i + 1) * (n // 8)], axis=1,
                 keepdims=True) for i in range(8)]
    return ((p[0] + p[1]) + (p[2] + p[3])) + ((p[4] + p[5]) + (p[6] + p[7]))


def _wta_kernel(x_ref, w_ref, b_ref, out_ref, t_ref, th_ref):
    step = pl.program_id(0)
    t_blk = jax.lax.dot_general(
        x_ref[...], w_ref[...],
        (((1,), (1,)), ((), ())),
        preferred_element_type=jnp.float32,
    ) + b_ref[...]
    t_ref[:, pl.ds(step * _BLOCK_N, _BLOCK_N)] = t_blk

    @pl.when(step == _NBLK - 1)
    def _():
        t = t_ref[...]
        msb = jnp.int32(-2147483648)

        prefix_u = jnp.zeros((_B, 1), jnp.int32)
        for bit in range(31, -1, -1):
            bitval = (1 << bit) if bit < 31 else -(1 << 31)
            cand_u = prefix_u | jnp.int32(bitval)
            cnt = _count_ge_tree(t, _key_float(cand_u ^ msb))
            prefix_u = jnp.where(cnt >= _K, cand_u, prefix_u)
        th_ref[...] = jnp.broadcast_to(prefix_u ^ msb, (_B, 128))

        tf = _key_float(th_ref[:, :1])
        ge = t >= tf
        n_ge = _count_ge_tree(t, tf)
        col = jax.lax.broadcasted_iota(jnp.int32, (_B, _OUT), 1)
        neg = jnp.float32(-jnp.inf)

        pooled = jnp.max(jnp.where(ge, t, neg), axis=0, keepdims=True)
        out_ref[...] = jnp.where(pooled == neg, jnp.float32(0.0), pooled)

        @pl.when(jnp.any(n_ge > _K))
        def _tie():
            gt = t > tf
            eq = ge & jnp.logical_not(gt)
            r = _K - (n_ge - jnp.sum(eq.astype(jnp.int32), axis=1,
                                     keepdims=True))
            mpref = jnp.zeros((_B, 1), jnp.int32)
            for bit in range(15, -1, -1):
                cand = mpref | jnp.int32(1 << bit)
                cntc = jnp.sum((eq & (col < cand)).astype(jnp.int32),
                               axis=1, keepdims=True)
                mpref = jnp.where(cntc <= r, cand, mpref)
            accept = gt | (eq & (col < mpref))
            pooled2 = jnp.max(jnp.where(accept, t, neg), axis=0,
                              keepdims=True)
            out_ref[...] = jnp.where(pooled2 == neg, jnp.float32(0.0),
                                     pooled2)


def kernel(inputs, W, b):
    out = pl.pallas_call(
        _wta_kernel,
        grid=(_NBLK,),
        in_specs=[
            pl.BlockSpec((_B, _IN), lambda i: (0, 0)),
            pl.BlockSpec((_BLOCK_N, _IN), lambda i: (i, 0)),
            pl.BlockSpec((1, _BLOCK_N), lambda i: (0, i)),
        ],
        out_specs=pl.BlockSpec((1, _OUT), lambda i: (0, 0)),
        out_shape=jax.ShapeDtypeStruct((1, _OUT), jnp.float32),
        scratch_shapes=[
            pltpu.VMEM((_B, _OUT), jnp.float32),
            pltpu.VMEM((_B, 128), jnp.int32),
        ],
    )(inputs, W, b.reshape(1, _OUT))
    return out.reshape(_OUT)
